# Initial kernel scaffold; baseline (speedup 1.0000x reference)
import jax
import jax.numpy as jnp
from jax.experimental import pallas as pl


def kernel(x, edge_index, batch, W_emb, b_emb, Ws, Wn, bl, gamma, beta, fcW0, fcb0, fcW1, fcb1, fcW2, fcb2):
    raise NotImplementedError("write your pallas kernel here")


# speedup vs baseline: 4.4715x; 4.4715x over previous
import functools

import jax
import jax.numpy as jnp
from jax import lax
from jax.experimental import pallas as pl
from jax.experimental.pallas import tpu as pltpu
from jax.experimental.pallas import tpu_sc as plsc

N = 10000
E = 320000
D = 128
H = 128
G = 64
C = 10
L = 3

NC = 2
NS = 16
ND = N // NC
ACCR = 5120
K = 80
EPT = E // NS
NCH = EPT // K
EPD = E // (NC * NS)
NCHD = EPD // K
ZRA = 8
SBW = 5
WRP = ND // SBW
SBD = 10
SRD = N // SBD
ZRD = 8


@functools.cache
def _sc_mesh():
    return plsc.VectorSubcoreMesh(core_axis_name="c", subcore_axis_name="s",
                                  num_cores=NC, num_subcores=NS)


def _dot(a, b):
    return lax.dot_general(a, b, (((1,), (0,)), ((), ())),
                           precision=lax.Precision.HIGHEST,
                           preferred_element_type=jnp.float32)


def _agg_body(h_hbm, src_hbm, dst_hbm, out_hbm,
              acc, srcv, dstv, dsta, dstb, bufa, bufb, zbuf, sem_a, sem_b):
    c = lax.axis_index("c")
    s = lax.axis_index("s")

    @pl.loop(0, ZRA)
    def _(r):
        for q in range(H // 16):
            zbuf[r, pl.ds(16 * q, 16)] = jnp.zeros((16,), jnp.float32)
    z0 = s * (ACCR // NS)
    for t in range((ACCR // NS) // ZRA):
        pltpu.sync_copy(zbuf, acc.at[pl.ds(z0 + t * ZRA, ZRA)])

    pltpu.sync_copy(src_hbm.at[s], srcv)
    pltpu.sync_copy(dst_hbm.at[s], dstv)
    lo = c * ND
    trash = ND + s

    def remap(chunk, idx_ref):
        for q in range(K // 16):
            v = dstv[chunk, pl.ds(16 * q, 16)] - lo
            ok = (v >= 0) & (v < ND)
            idx_ref[pl.ds(16 * q, 16)] = jnp.where(ok, v, trash)

    plsc.subcore_barrier()

    @pl.loop(0, NCH // 2)
    def _(jj):
        c0 = 2 * jj
        cpa = pltpu.async_copy(h_hbm.at[srcv.at[c0]], bufa, sem_a)
        cpb = pltpu.async_copy(h_hbm.at[srcv.at[c0 + 1]], bufb, sem_b)
        remap(c0, dsta)
        cpa.wait()
        pltpu.sync_copy(bufa, acc.at[dsta], add=True)
        remap(c0 + 1, dstb)
        cpb.wait()
        pltpu.sync_copy(bufb, acc.at[dstb], add=True)

    plsc.subcore_barrier()

    @pl.when(s < SBW)
    def _():
        pltpu.sync_copy(acc.at[pl.ds(s * WRP, WRP)],
                        out_hbm.at[pl.ds(c * ND + s * WRP, WRP)])


@functools.cache
def _agg_kernel():
    return pl.kernel(
        _agg_body,
        out_type=jax.ShapeDtypeStruct((N, H), jnp.float32),
        mesh=_sc_mesh(),
        scratch_types=[
            pltpu.VMEM_SHARED((ACCR, H), jnp.float32),
            pltpu.VMEM((NCH, K), jnp.int32),
            pltpu.VMEM((NCH, K), jnp.int32),
            pltpu.VMEM((K,), jnp.int32),
            pltpu.VMEM((K,), jnp.int32),
            pltpu.VMEM((K, H), jnp.float32),
            pltpu.VMEM((K, H), jnp.float32),
            pltpu.VMEM((ZRA, H), jnp.float32),
            pltpu.SemaphoreType.DMA,
            pltpu.SemaphoreType.DMA,
        ])


def _deg_body(dst_hbm, out_hbm, accd, dstv, dstidx, ones, zbuf):
    c = lax.axis_index("c")
    s = lax.axis_index("s")
    r0 = s * SRD

    @pl.when(s < SBD)
    def _():
        @pl.loop(0, ZRD)
        def _(r):
            for q in range(H // 16):
                zbuf[r, pl.ds(16 * q, 16)] = jnp.zeros((16,), jnp.float32)
        for t in range(SRD // ZRD):
            pltpu.sync_copy(zbuf, accd.at[pl.ds(r0 + t * ZRD, ZRD)])

    @pl.loop(0, K)
    def _(r):
        for q in range(H // 16):
            ones[r, pl.ds(16 * q, 16)] = jnp.full((16,), 1.0, jnp.float32)

    pltpu.sync_copy(dst_hbm.at[c, s], dstv)
    plsc.subcore_barrier()

    @pl.loop(0, NCHD)
    def _(j):
        for q in range(K // 16):
            dstidx[pl.ds(16 * q, 16)] = dstv[j, pl.ds(16 * q, 16)]
        pltpu.sync_copy(ones, accd.at[dstidx], add=True)

    plsc.subcore_barrier()

    @pl.when(s < SBD)
    def _():
        pltpu.sync_copy(accd.at[pl.ds(r0, SRD)],
                        out_hbm.at[c, pl.ds(r0, SRD)])


@functools.cache
def _deg_kernel():
    return pl.kernel(
        _deg_body,
        out_type=jax.ShapeDtypeStruct((NC, N, H), jnp.float32),
        mesh=_sc_mesh(),
        scratch_types=[
            pltpu.VMEM_SHARED((N, H), jnp.float32),
            pltpu.VMEM((NCHD, K), jnp.int32),
            pltpu.VMEM((K,), jnp.int32),
            pltpu.VMEM((K, H), jnp.float32),
            pltpu.VMEM((ZRD, H), jnp.float32),
        ])


def _pre_body(x_ref, w_ref, b_ref, batch_ref, degp_ref,
              h0_ref, dinv_ref, p_ref, cinv_ref):
    h0_ref[...] = _dot(x_ref[...], w_ref[...]) + b_ref[...]
    deg = degp_ref[0, :, 0:1] + degp_ref[1, :, 0:1]
    dinv_ref[...] = 1.0 / jnp.maximum(deg, 1.0)
    gids = lax.broadcasted_iota(jnp.int32, (G, N), 0)
    p = (batch_ref[...] == gids).astype(jnp.float32)
    p_ref[...] = p
    counts = jnp.sum(p, axis=1, keepdims=True)
    cinv_ref[...] = 1.0 / jnp.maximum(counts, 1.0)


_pre = pl.pallas_call(
    _pre_body,
    out_shape=[
        jax.ShapeDtypeStruct((N, H), jnp.float32),
        jax.ShapeDtypeStruct((N, 1), jnp.float32),
        jax.ShapeDtypeStruct((G, N), jnp.float32),
        jax.ShapeDtypeStruct((G, 1), jnp.float32),
    ])


def _layer_body(h_ref, agg_ref, dinv_ref, p_ref, cinv_ref,
                ws_ref, wn_ref, blb_ref, g_ref, b_ref,
                ho_ref, feat_ref):
    h = h_ref[...]
    a = agg_ref[...] * dinv_ref[...]
    hn = _dot(h, ws_ref[...]) + _dot(a, wn_ref[...]) + blb_ref[...]
    mu = jnp.mean(hn, axis=0, keepdims=True)
    dlt = hn - mu
    var = jnp.mean(dlt * dlt, axis=0, keepdims=True)
    hn = g_ref[...] * dlt * lax.rsqrt(var + 1e-5) + b_ref[...]
    ho = jnp.maximum(hn, 0.0) + h
    ho_ref[...] = ho
    feat_ref[...] = _dot(p_ref[...], ho) * cinv_ref[...]


_layer = pl.pallas_call(
    _layer_body,
    out_shape=[
        jax.ShapeDtypeStruct((N, H), jnp.float32),
        jax.ShapeDtypeStruct((G, H), jnp.float32),
    ])


def _readout_body(f_ref, w0_ref, b0_ref, w1_ref, b1_ref, w2_ref, b2_ref,
                  out_ref):
    for l in range(L):
        y = jnp.maximum(_dot(f_ref[l], w0_ref[l]) + b0_ref[l], 0.0)
        y = jnp.maximum(_dot(y, w1_ref[l]) + b1_ref[l], 0.0)
        out_ref[l, :, :] = _dot(y, w2_ref[l]) + b2_ref[l]


_readout = pl.pallas_call(
    _readout_body,
    out_shape=jax.ShapeDtypeStruct((L, G, C), jnp.float32))


def kernel(x, edge_index, batch, W_emb, b_emb, Ws, Wn, bl, gamma, beta,
           fcW0, fcb0, fcW1, fcb1, fcW2, fcb2):
    src = edge_index[0].reshape(NS, NCH, K)
    dst = edge_index[1].reshape(NS, NCH, K)
    dstd = edge_index[1].reshape(NC, NS, NCHD, K)

    degp = _deg_kernel()(dstd)
    h, dinv, p, cinv = _pre(x, W_emb, b_emb.reshape(1, H),
                            batch.reshape(1, N), degp)

    feats = []
    for l in range(L):
        agg = _agg_kernel()(h, src, dst)
        h, feat = _layer(h, agg, dinv, p, cinv, Ws[l], Wn[l],
                         bl[l].reshape(1, H), gamma[l].reshape(1, H),
                         beta[l].reshape(1, H))
        feats.append(feat)

    outs = _readout(jnp.stack(feats), fcW0, fcb0.reshape(L, 1, H // 2),
                    fcW1, fcb1.reshape(L, 1, H // 4),
                    fcW2, fcb2.reshape(L, 1, C))
    return (tuple(outs[l] for l in range(L)), tuple(feats))
